# Initial kernel scaffold; baseline (speedup 1.0000x reference)
import jax
import jax.numpy as jnp
from jax.experimental import pallas as pl


def kernel(x, edge_index, ln0_w, ln0_b, W1, b1, ln1_w, ln1_b, W2, b2, ln2_w, ln2_b):
    raise NotImplementedError("write your pallas kernel here")


# speedup vs baseline: 30.6917x; 30.6917x over previous
import functools

import jax
import jax.numpy as jnp
from jax import lax
from jax.experimental import pallas as pl
from jax.experimental.pallas import tpu as pltpu
from jax.experimental.pallas import tpu_sc as plsc

N = 10000
E = 320000
D = 128

NC = 2
NS = 16
NW = NC * NS

EPW = E // NW
CHUNK = 80
NCHUNK = 125
EPAD = NW * NCHUNK * CHUNK - E
NPAD = 10240
RPT = NPAD // NS

_mesh = plsc.VectorSubcoreMesh(core_axis_name="c", subcore_axis_name="s")


@functools.partial(
    pl.kernel,
    out_type=jax.ShapeDtypeStruct((NC, NS, N + 240), jnp.float32),
    mesh=_mesh,
    compiler_params=pltpu.CompilerParams(needs_layout_passes=False),
    scratch_types=[
        pltpu.VMEM((EPW,), jnp.int32),
        pltpu.VMEM((N + 240,), jnp.float32),
    ],
)
def _sc_degree(dst_hbm, out_hbm, dst_v, hist):
    c = lax.axis_index("c")
    s = lax.axis_index("s")
    wid = s * NC + c

    zero16 = jnp.zeros((16,), jnp.float32)

    def zb(j, carry):
        hist[pl.ds(j * 16, 16)] = zero16
        return carry

    lax.fori_loop(0, (N + 240) // 16, zb, 0)

    pltpu.sync_copy(dst_hbm.at[pl.ds(wid * EPW, EPW)], dst_v)

    ones16 = jnp.ones((16,), jnp.float32)

    def step(j, carry):
        idx = dst_v[pl.ds(j * 16, 16)]
        plsc.addupdate_scatter(hist, [idx], ones16)
        return carry

    lax.fori_loop(0, EPW // 16, step, 0)

    pltpu.sync_copy(hist, out_hbm.at[c, s])


@functools.partial(
    pl.kernel,
    out_type=jax.ShapeDtypeStruct((NC, NPAD, D), jnp.float32),
    mesh=_mesh,
    compiler_params=pltpu.CompilerParams(needs_layout_passes=False),
    scratch_types=[
        pltpu.VMEM((NCHUNK, CHUNK), jnp.int32),
        pltpu.VMEM((1, CHUNK), jnp.int32),
        pltpu.VMEM((1, CHUNK), jnp.int32),
        pltpu.VMEM((CHUNK, D), jnp.float32),
        pltpu.VMEM((CHUNK, D), jnp.float32),
        pltpu.VMEM_SHARED((NPAD, D), jnp.float32),
        pltpu.SemaphoreType.DMA,
        pltpu.SemaphoreType.DMA,
        pltpu.SemaphoreType.DMA,
        pltpu.SemaphoreType.DMA,
    ],
)
def _sc_edge_scatter(y_hbm, srcg_hbm, dstg_hbm, out_hbm,
                     src2d, db0, db1, rows0, rows1, acc,
                     semg0, semg1, semd0, semd1):
    c = lax.axis_index("c")
    s = lax.axis_index("s")
    wid = s * NC + c

    zero16 = jnp.zeros((16,), jnp.float32)

    def zb(j, carry):
        for k in range(8):
            rows0[j, pl.ds(k * 16, 16)] = zero16
        return carry

    lax.fori_loop(0, CHUNK, zb, 0)

    for m in range(RPT // 80):
        pltpu.sync_copy(rows0.at[pl.ds(0, 80)],
                        acc.at[pl.ds(s * RPT + m * 80, 80)])

    pltpu.sync_copy(srcg_hbm.at[wid], src2d)

    plsc.subcore_barrier()

    def gather(i, buf, semg, db, semd):
        pltpu.async_copy(y_hbm.at[src2d.at[i]], buf, semg)
        pltpu.async_copy(dstg_hbm.at[wid, i], db, semd)

    def gwait(buf, semg, db, semd):
        pltpu.make_async_copy(y_hbm.at[pl.ds(0, CHUNK)], buf, semg).wait()
        pltpu.make_async_copy(dstg_hbm.at[wid, 0], db, semd).wait()

    def scatter(db, buf):
        pltpu.sync_copy(buf, acc.at[db.at[0]], add=True)

    gather(0, rows0, semg0, db0, semd0)
    gather(1, rows1, semg1, db1, semd1)

    def step(j, carry):
        i0 = 2 * j
        gwait(rows0, semg0, db0, semd0)
        scatter(db0, rows0)
        gather(i0 + 2, rows0, semg0, db0, semd0)
        gwait(rows1, semg1, db1, semd1)
        scatter(db1, rows1)
        gather(i0 + 3, rows1, semg1, db1, semd1)
        return carry

    lax.fori_loop(0, (NCHUNK - 3) // 2, step, 0)

    if NCHUNK % 2:
        gwait(rows0, semg0, db0, semd0)
        scatter(db0, rows0)
        gather(NCHUNK - 1, rows0, semg0, db0, semd0)
        gwait(rows1, semg1, db1, semd1)
        scatter(db1, rows1)
        gwait(rows0, semg0, db0, semd0)
        scatter(db0, rows0)
    else:
        gwait(rows0, semg0, db0, semd0)
        scatter(db0, rows0)
        gather(NCHUNK - 2, rows0, semg0, db0, semd0)
        gwait(rows1, semg1, db1, semd1)
        scatter(db1, rows1)
        gather(NCHUNK - 1, rows1, semg1, db1, semd1)
        gwait(rows0, semg0, db0, semd0)
        scatter(db0, rows0)
        gwait(rows1, semg1, db1, semd1)
        scatter(db1, rows1)

    plsc.subcore_barrier()

    pltpu.sync_copy(acc.at[pl.ds(s * RPT, RPT)],
                    out_hbm.at[c, pl.ds(s * RPT, RPT)])


BLK = 2000
GRID = N // BLK


def _ln(h, w, b):
    mu = jnp.mean(h, axis=1, keepdims=True)
    var = jnp.mean((h - mu) * (h - mu), axis=1, keepdims=True)
    return (h - mu) * lax.rsqrt(var + 1e-5) * w + b


def _dinv(degt):
    return lax.rsqrt(jnp.sum(degt, axis=1, keepdims=True) + 1.0)


def _tc_pre_body(x_ref, degt_ref, lnw_ref, lnb_ref, w_ref, o_ref):
    h = _ln(x_ref[...], lnw_ref[...], lnb_ref[...])
    o_ref[...] = _dinv(degt_ref[...]) * jnp.dot(
        h, w_ref[...], preferred_element_type=jnp.float32)


def _tc_mid_body(acca_ref, accb_ref, y_ref, degt_ref, b_ref,
                 lnw_ref, lnb_ref, w_ref, o_ref):
    dinv = _dinv(degt_ref[...])
    t = dinv * (acca_ref[0] + accb_ref[0] + y_ref[...]) + b_ref[...]
    t = jnp.maximum(t, 0.0)
    h = _ln(t, lnw_ref[...], lnb_ref[...])
    o_ref[...] = dinv * jnp.dot(h, w_ref[...],
                                preferred_element_type=jnp.float32)


def _tc_post_body(acca_ref, accb_ref, y_ref, degt_ref, b_ref,
                  lnw_ref, lnb_ref, o_ref):
    dinv = _dinv(degt_ref[...])
    t = dinv * (acca_ref[0] + accb_ref[0] + y_ref[...]) + b_ref[...]
    t = jnp.maximum(t, 0.0)
    h = _ln(t, lnw_ref[...], lnb_ref[...])

    i = pl.program_id(0)

    @pl.when(i == 0)
    def _():
        o_ref[...] = jnp.zeros_like(o_ref)

    o_ref[...] += jnp.sum(h, axis=0, keepdims=True) * (1.0 / N)


_row_spec = pl.BlockSpec((BLK, D), lambda i: (i, 0))
_deg_spec = pl.BlockSpec((BLK, NW), lambda i: (i, 0))
_vec_spec = pl.BlockSpec((1, D), lambda i: (0, 0))
_mat_spec = pl.BlockSpec((D, D), lambda i: (0, 0))
_acca_spec = pl.BlockSpec((1, BLK, D), lambda i: (0, i, 0))
_accb_spec = pl.BlockSpec((1, BLK, D), lambda i: (1, i, 0))


def _row_out(shape):
    return dict(out_specs=_row_spec,
                out_shape=jax.ShapeDtypeStruct(shape, jnp.float32))


def kernel(x, edge_index, ln0_w, ln0_b, W1, b1, ln1_w, ln1_b, W2, b2,
           ln2_w, ln2_b):
    src = edge_index[0]
    dst = edge_index[1]
    src_p = jnp.concatenate([src, jnp.zeros((EPAD,), jnp.int32)])
    dst_p = jnp.concatenate(
        [dst, N + (jnp.arange(EPAD, dtype=jnp.int32) % (NPAD - N))])
    srcg = src_p.reshape(NW, NCHUNK, CHUNK)
    dstg = dst_p.reshape(NW, NCHUNK, 1, CHUNK)

    ln0_w = ln0_w.reshape(1, D)
    ln0_b = ln0_b.reshape(1, D)
    ln1_w = ln1_w.reshape(1, D)
    ln1_b = ln1_b.reshape(1, D)
    ln2_w = ln2_w.reshape(1, D)
    ln2_b = ln2_b.reshape(1, D)
    b1 = b1.reshape(1, D)
    b2 = b2.reshape(1, D)

    deg_parts = _sc_degree(dst)
    degt = deg_parts.reshape(NW, N + 240).T[:N]

    y1 = pl.pallas_call(
        _tc_pre_body,
        grid=(GRID,),
        in_specs=[_row_spec, _deg_spec, _vec_spec, _vec_spec, _mat_spec],
        **_row_out((N, D)),
    )(x, degt, ln0_w, ln0_b, W1)

    acc1 = _sc_edge_scatter(y1, srcg, dstg)

    y2 = pl.pallas_call(
        _tc_mid_body,
        grid=(GRID,),
        in_specs=[_acca_spec, _accb_spec, _row_spec, _deg_spec,
                  _vec_spec, _vec_spec, _vec_spec, _mat_spec],
        **_row_out((N, D)),
    )(acc1, acc1, y1, degt, b1, ln1_w, ln1_b, W2)

    acc2 = _sc_edge_scatter(y2, srcg, dstg)

    out = pl.pallas_call(
        _tc_post_body,
        grid=(GRID,),
        in_specs=[_acca_spec, _accb_spec, _row_spec, _deg_spec,
                  _vec_spec, _vec_spec, _vec_spec],
        out_specs=pl.BlockSpec((1, D), lambda i: (0, 0)),
        out_shape=jax.ShapeDtypeStruct((1, D), jnp.float32),
    )(acc2, acc2, y2, degt, b2, ln2_w, ln2_b)

    return out


# speedup vs baseline: 31.0211x; 1.0107x over previous
import functools

import jax
import jax.numpy as jnp
from jax import lax
from jax.experimental import pallas as pl
from jax.experimental.pallas import tpu as pltpu
from jax.experimental.pallas import tpu_sc as plsc

N = 10000
E = 320000
D = 128

NC = 2
NS = 16
NW = NC * NS

EPW = E // NW
CHUNK = 80
NCHUNK = 125
EPAD = NW * NCHUNK * CHUNK - E
NPAD = 10240
RPT = NPAD // NS

_mesh = plsc.VectorSubcoreMesh(core_axis_name="c", subcore_axis_name="s")


@functools.partial(
    pl.kernel,
    out_type=jax.ShapeDtypeStruct((NC, NS, N + 240), jnp.float32),
    mesh=_mesh,
    compiler_params=pltpu.CompilerParams(needs_layout_passes=False),
    scratch_types=[
        pltpu.VMEM((EPW,), jnp.int32),
        pltpu.VMEM((N + 240,), jnp.float32),
    ],
)
def _sc_degree(dst_hbm, out_hbm, dst_v, hist):
    c = lax.axis_index("c")
    s = lax.axis_index("s")
    wid = s * NC + c

    zero16 = jnp.zeros((16,), jnp.float32)

    def zb(j, carry):
        hist[pl.ds(j * 16, 16)] = zero16
        return carry

    lax.fori_loop(0, (N + 240) // 16, zb, 0)

    pltpu.sync_copy(dst_hbm.at[pl.ds(wid * EPW, EPW)], dst_v)

    ones16 = jnp.ones((16,), jnp.float32)

    def step(j, carry):
        idx = dst_v[pl.ds(j * 16, 16)]
        plsc.addupdate_scatter(hist, [idx], ones16)
        return carry

    lax.fori_loop(0, EPW // 16, step, 0)

    pltpu.sync_copy(hist, out_hbm.at[c, s])


@functools.partial(
    pl.kernel,
    out_type=jax.ShapeDtypeStruct((NC, NPAD, D), jnp.float32),
    mesh=_mesh,
    compiler_params=pltpu.CompilerParams(needs_layout_passes=False),
    scratch_types=[
        pltpu.VMEM((NCHUNK, CHUNK), jnp.int32),
        pltpu.VMEM((1, CHUNK), jnp.int32),
        pltpu.VMEM((1, CHUNK), jnp.int32),
        pltpu.VMEM((CHUNK, D), jnp.float32),
        pltpu.VMEM((CHUNK, D), jnp.float32),
        pltpu.VMEM_SHARED((NPAD, D), jnp.float32),
        pltpu.SemaphoreType.DMA,
        pltpu.SemaphoreType.DMA,
        pltpu.SemaphoreType.DMA,
        pltpu.SemaphoreType.DMA,
    ],
)
def _sc_edge_scatter(y_hbm, srcg_hbm, dstg_hbm, out_hbm,
                     src2d, db0, db1, rows0, rows1, acc,
                     semg0, semg1, semd0, semd1):
    c = lax.axis_index("c")
    s = lax.axis_index("s")
    wid = s * NC + c

    zero16 = jnp.zeros((16,), jnp.float32)

    def zb(j, carry):
        for k in range(8):
            rows0[j, pl.ds(k * 16, 16)] = zero16
        return carry

    lax.fori_loop(0, CHUNK, zb, 0)

    for m in range(RPT // 80):
        pltpu.sync_copy(rows0.at[pl.ds(0, 80)],
                        acc.at[pl.ds(s * RPT + m * 80, 80)])

    pltpu.sync_copy(srcg_hbm.at[wid], src2d)

    plsc.subcore_barrier()

    def gather(i, buf, semg, db, semd):
        pltpu.async_copy(y_hbm.at[src2d.at[i]], buf, semg)
        pltpu.async_copy(dstg_hbm.at[wid, i], db, semd)

    def gwait(buf, semg, db, semd):
        pltpu.make_async_copy(y_hbm.at[pl.ds(0, CHUNK)], buf, semg).wait()
        pltpu.make_async_copy(dstg_hbm.at[wid, 0], db, semd).wait()

    def scatter(db, buf):
        pltpu.sync_copy(buf, acc.at[db.at[0]], add=True)

    gather(0, rows0, semg0, db0, semd0)
    gather(1, rows1, semg1, db1, semd1)

    def step(j, carry):
        i0 = 2 * j
        gwait(rows0, semg0, db0, semd0)
        scatter(db0, rows0)
        gather(i0 + 2, rows0, semg0, db0, semd0)
        gwait(rows1, semg1, db1, semd1)
        scatter(db1, rows1)
        gather(i0 + 3, rows1, semg1, db1, semd1)
        return carry

    lax.fori_loop(0, (NCHUNK - 3) // 2, step, 0)

    if NCHUNK % 2:
        gwait(rows0, semg0, db0, semd0)
        scatter(db0, rows0)
        gather(NCHUNK - 1, rows0, semg0, db0, semd0)
        gwait(rows1, semg1, db1, semd1)
        scatter(db1, rows1)
        gwait(rows0, semg0, db0, semd0)
        scatter(db0, rows0)
    else:
        gwait(rows0, semg0, db0, semd0)
        scatter(db0, rows0)
        gather(NCHUNK - 2, rows0, semg0, db0, semd0)
        gwait(rows1, semg1, db1, semd1)
        scatter(db1, rows1)
        gather(NCHUNK - 1, rows1, semg1, db1, semd1)
        gwait(rows0, semg0, db0, semd0)
        scatter(db0, rows0)
        gwait(rows1, semg1, db1, semd1)
        scatter(db1, rows1)

    plsc.subcore_barrier()

    pltpu.sync_copy(acc.at[pl.ds(s * RPT, RPT)],
                    out_hbm.at[c, pl.ds(s * RPT, RPT)])


BLK = 5000
GRID = N // BLK


def _ln(h, w, b):
    mu = jnp.mean(h, axis=1, keepdims=True)
    var = jnp.mean((h - mu) * (h - mu), axis=1, keepdims=True)
    return (h - mu) * lax.rsqrt(var + 1e-5) * w + b


def _dinv(degt):
    return lax.rsqrt(jnp.sum(degt, axis=1, keepdims=True) + 1.0)


def _tc_pre_body(x_ref, degt_ref, lnw_ref, lnb_ref, w_ref, o_ref):
    h = _ln(x_ref[...], lnw_ref[...], lnb_ref[...])
    o_ref[...] = _dinv(degt_ref[...]) * jnp.dot(
        h, w_ref[...], preferred_element_type=jnp.float32)


def _tc_mid_body(acca_ref, accb_ref, y_ref, degt_ref, b_ref,
                 lnw_ref, lnb_ref, w_ref, o_ref):
    dinv = _dinv(degt_ref[...])
    t = dinv * (acca_ref[0] + accb_ref[0] + y_ref[...]) + b_ref[...]
    t = jnp.maximum(t, 0.0)
    h = _ln(t, lnw_ref[...], lnb_ref[...])
    o_ref[...] = dinv * jnp.dot(h, w_ref[...],
                                preferred_element_type=jnp.float32)


def _tc_post_body(acca_ref, accb_ref, y_ref, degt_ref, b_ref,
                  lnw_ref, lnb_ref, o_ref):
    dinv = _dinv(degt_ref[...])
    t = dinv * (acca_ref[0] + accb_ref[0] + y_ref[...]) + b_ref[...]
    t = jnp.maximum(t, 0.0)
    h = _ln(t, lnw_ref[...], lnb_ref[...])

    i = pl.program_id(0)

    @pl.when(i == 0)
    def _():
        o_ref[...] = jnp.zeros_like(o_ref)

    o_ref[...] += jnp.sum(h, axis=0, keepdims=True) * (1.0 / N)


_row_spec = pl.BlockSpec((BLK, D), lambda i: (i, 0))
_deg_spec = pl.BlockSpec((BLK, NW), lambda i: (i, 0))
_vec_spec = pl.BlockSpec((1, D), lambda i: (0, 0))
_mat_spec = pl.BlockSpec((D, D), lambda i: (0, 0))
_acca_spec = pl.BlockSpec((1, BLK, D), lambda i: (0, i, 0))
_accb_spec = pl.BlockSpec((1, BLK, D), lambda i: (1, i, 0))


def _row_out(shape):
    return dict(out_specs=_row_spec,
                out_shape=jax.ShapeDtypeStruct(shape, jnp.float32))


def kernel(x, edge_index, ln0_w, ln0_b, W1, b1, ln1_w, ln1_b, W2, b2,
           ln2_w, ln2_b):
    src = edge_index[0]
    dst = edge_index[1]
    src_p = jnp.concatenate([src, jnp.zeros((EPAD,), jnp.int32)])
    dst_p = jnp.concatenate(
        [dst, N + (jnp.arange(EPAD, dtype=jnp.int32) % (NPAD - N))])
    srcg = src_p.reshape(NW, NCHUNK, CHUNK)
    dstg = dst_p.reshape(NW, NCHUNK, 1, CHUNK)

    ln0_w = ln0_w.reshape(1, D)
    ln0_b = ln0_b.reshape(1, D)
    ln1_w = ln1_w.reshape(1, D)
    ln1_b = ln1_b.reshape(1, D)
    ln2_w = ln2_w.reshape(1, D)
    ln2_b = ln2_b.reshape(1, D)
    b1 = b1.reshape(1, D)
    b2 = b2.reshape(1, D)

    deg_parts = _sc_degree(dst)
    degt = deg_parts.reshape(NW, N + 240).T[:N]

    y1 = pl.pallas_call(
        _tc_pre_body,
        grid=(GRID,),
        in_specs=[_row_spec, _deg_spec, _vec_spec, _vec_spec, _mat_spec],
        **_row_out((N, D)),
    )(x, degt, ln0_w, ln0_b, W1)

    acc1 = _sc_edge_scatter(y1, srcg, dstg)

    y2 = pl.pallas_call(
        _tc_mid_body,
        grid=(GRID,),
        in_specs=[_acca_spec, _accb_spec, _row_spec, _deg_spec,
                  _vec_spec, _vec_spec, _vec_spec, _mat_spec],
        **_row_out((N, D)),
    )(acc1, acc1, y1, degt, b1, ln1_w, ln1_b, W2)

    acc2 = _sc_edge_scatter(y2, srcg, dstg)

    out = pl.pallas_call(
        _tc_post_body,
        grid=(GRID,),
        in_specs=[_acca_spec, _accb_spec, _row_spec, _deg_spec,
                  _vec_spec, _vec_spec, _vec_spec],
        out_specs=pl.BlockSpec((1, D), lambda i: (0, 0)),
        out_shape=jax.ShapeDtypeStruct((1, D), jnp.float32),
    )(acc2, acc2, y2, degt, b2, ln2_w, ln2_b)

    return out


# speedup vs baseline: 31.3162x; 1.0095x over previous
import functools

import jax
import jax.numpy as jnp
from jax import lax
from jax.experimental import pallas as pl
from jax.experimental.pallas import tpu as pltpu
from jax.experimental.pallas import tpu_sc as plsc

N = 10000
E = 320000
D = 128

NC = 2
NS = 16
NW = NC * NS

EPW = E // NW
CHUNK = 80
NCHUNK = 125
EPAD = NW * NCHUNK * CHUNK - E
NPAD = 10240
RPT = NPAD // NS

_mesh = plsc.VectorSubcoreMesh(core_axis_name="c", subcore_axis_name="s")


@functools.partial(
    pl.kernel,
    out_type=jax.ShapeDtypeStruct((NC, NS, N + 240), jnp.float32),
    mesh=_mesh,
    compiler_params=pltpu.CompilerParams(needs_layout_passes=False),
    scratch_types=[
        pltpu.VMEM((EPW,), jnp.int32),
        pltpu.VMEM((N + 240,), jnp.float32),
    ],
)
def _sc_degree(dst_hbm, out_hbm, dst_v, hist):
    c = lax.axis_index("c")
    s = lax.axis_index("s")
    wid = s * NC + c

    zero16 = jnp.zeros((16,), jnp.float32)

    def zb(j, carry):
        hist[pl.ds(j * 16, 16)] = zero16
        return carry

    lax.fori_loop(0, (N + 240) // 16, zb, 0)

    pltpu.sync_copy(dst_hbm.at[pl.ds(wid * EPW, EPW)], dst_v)

    ones16 = jnp.ones((16,), jnp.float32)

    def step(j, carry):
        idx = dst_v[pl.ds(j * 16, 16)]
        plsc.addupdate_scatter(hist, [idx], ones16)
        return carry

    lax.fori_loop(0, EPW // 16, step, 0)

    pltpu.sync_copy(hist, out_hbm.at[c, s])


@functools.partial(
    pl.kernel,
    out_type=jax.ShapeDtypeStruct((NC, NPAD, D), jnp.float32),
    mesh=_mesh,
    compiler_params=pltpu.CompilerParams(needs_layout_passes=False),
    scratch_types=[
        pltpu.VMEM((2, CHUNK), jnp.int32),
        pltpu.VMEM((2, CHUNK), jnp.int32),
        pltpu.VMEM((2, CHUNK), jnp.int32),
        pltpu.VMEM((CHUNK, D), jnp.float32),
        pltpu.VMEM((CHUNK, D), jnp.float32),
        pltpu.VMEM((CHUNK, D), jnp.float32),
        pltpu.VMEM_SHARED((NPAD, D), jnp.float32),
        pltpu.SemaphoreType.DMA,
        pltpu.SemaphoreType.DMA,
        pltpu.SemaphoreType.DMA,
        pltpu.SemaphoreType.DMA,
        pltpu.SemaphoreType.DMA,
        pltpu.SemaphoreType.DMA,
    ],
)
def _sc_edge_scatter(y_hbm, sdg_hbm, out_hbm,
                     sd0, sd1, sd2, rows0, rows1, rows2, acc,
                     seml0, seml1, seml2, semg0, semg1, semg2):
    c = lax.axis_index("c")
    s = lax.axis_index("s")
    wid = s * NC + c

    zero16 = jnp.zeros((16,), jnp.float32)

    def zb(j, carry):
        for k in range(8):
            rows0[j, pl.ds(k * 16, 16)] = zero16
        return carry

    lax.fori_loop(0, CHUNK, zb, 0)

    for m in range(RPT // 80):
        pltpu.sync_copy(rows0.at[pl.ds(0, 80)],
                        acc.at[pl.ds(s * RPT + m * 80, 80)])

    sds = (sd0, sd1, sd2)
    rowss = (rows0, rows1, rows2)
    semls = (seml0, seml1, seml2)
    semgs = (semg0, semg1, semg2)

    def idxload(i, b):
        pltpu.async_copy(sdg_hbm.at[wid, i], sds[b], semls[b])

    def lwait(b):
        pltpu.make_async_copy(sdg_hbm.at[wid, 0], sds[b], semls[b]).wait()

    def gather(i, b):
        pltpu.async_copy(y_hbm.at[sds[b].at[0]], rowss[b], semgs[b])

    def gwait(b):
        pltpu.make_async_copy(y_hbm.at[pl.ds(0, CHUNK)], rowss[b],
                              semgs[b]).wait()

    def scatter(b):
        pltpu.sync_copy(rowss[b], acc.at[sds[b].at[1]], add=True)

    idxload(0, 0)
    idxload(1, 1)
    idxload(2, 2)
    lwait(0)
    gather(0, 0)
    lwait(1)
    gather(1, 1)

    plsc.subcore_barrier()

    def chunk_steady(i, b):
        gwait(b)
        scatter(b)
        idxload(i + 3, b)
        b2 = (b + 2) % 3
        lwait(b2)
        gather(i + 2, b2)

    def step(m, carry):
        i = 3 * m
        chunk_steady(i, 0)
        chunk_steady(i + 1, 1)
        chunk_steady(i + 2, 2)
        return carry

    lax.fori_loop(0, (NCHUNK - 5) // 3, step, 0)

    t = NCHUNK - 5
    chunk_steady(t, 0)
    chunk_steady(t + 1, 1)
    gwait(2)
    scatter(2)
    lwait(1)
    gather(t + 4, 1)
    gwait(0)
    scatter(0)
    gwait(1)
    scatter(1)

    plsc.subcore_barrier()

    pltpu.sync_copy(acc.at[pl.ds(s * RPT, RPT)],
                    out_hbm.at[c, pl.ds(s * RPT, RPT)])


BLK = 5000
GRID = N // BLK


def _ln(h, w, b):
    mu = jnp.mean(h, axis=1, keepdims=True)
    var = jnp.mean((h - mu) * (h - mu), axis=1, keepdims=True)
    return (h - mu) * lax.rsqrt(var + 1e-5) * w + b


def _dinv(degt):
    return lax.rsqrt(jnp.sum(degt, axis=1, keepdims=True) + 1.0)


def _tc_pre_body(x_ref, degt_ref, lnw_ref, lnb_ref, w_ref, o_ref):
    h = _ln(x_ref[...], lnw_ref[...], lnb_ref[...])
    o_ref[...] = _dinv(degt_ref[...]) * jnp.dot(
        h, w_ref[...], preferred_element_type=jnp.float32)


def _tc_mid_body(acca_ref, accb_ref, y_ref, degt_ref, b_ref,
                 lnw_ref, lnb_ref, w_ref, o_ref):
    dinv = _dinv(degt_ref[...])
    t = dinv * (acca_ref[0] + accb_ref[0] + y_ref[...]) + b_ref[...]
    t = jnp.maximum(t, 0.0)
    h = _ln(t, lnw_ref[...], lnb_ref[...])
    o_ref[...] = dinv * jnp.dot(h, w_ref[...],
                                preferred_element_type=jnp.float32)


def _tc_post_body(acca_ref, accb_ref, y_ref, degt_ref, b_ref,
                  lnw_ref, lnb_ref, o_ref):
    dinv = _dinv(degt_ref[...])
    t = dinv * (acca_ref[0] + accb_ref[0] + y_ref[...]) + b_ref[...]
    t = jnp.maximum(t, 0.0)
    h = _ln(t, lnw_ref[...], lnb_ref[...])

    i = pl.program_id(0)

    @pl.when(i == 0)
    def _():
        o_ref[...] = jnp.zeros_like(o_ref)

    o_ref[...] += jnp.sum(h, axis=0, keepdims=True) * (1.0 / N)


_row_spec = pl.BlockSpec((BLK, D), lambda i: (i, 0))
_deg_spec = pl.BlockSpec((BLK, NW), lambda i: (i, 0))
_vec_spec = pl.BlockSpec((1, D), lambda i: (0, 0))
_mat_spec = pl.BlockSpec((D, D), lambda i: (0, 0))
_acca_spec = pl.BlockSpec((1, BLK, D), lambda i: (0, i, 0))
_accb_spec = pl.BlockSpec((1, BLK, D), lambda i: (1, i, 0))


def _row_out(shape):
    return dict(out_specs=_row_spec,
                out_shape=jax.ShapeDtypeStruct(shape, jnp.float32))


def kernel(x, edge_index, ln0_w, ln0_b, W1, b1, ln1_w, ln1_b, W2, b2,
           ln2_w, ln2_b):
    dst = edge_index[1]
    sdg = edge_index.reshape(2, NW, NCHUNK, CHUNK).transpose(1, 2, 0, 3)

    ln0_w = ln0_w.reshape(1, D)
    ln0_b = ln0_b.reshape(1, D)
    ln1_w = ln1_w.reshape(1, D)
    ln1_b = ln1_b.reshape(1, D)
    ln2_w = ln2_w.reshape(1, D)
    ln2_b = ln2_b.reshape(1, D)
    b1 = b1.reshape(1, D)
    b2 = b2.reshape(1, D)

    deg_parts = _sc_degree(dst)
    degt = deg_parts.reshape(NW, N + 240).T[:N]

    y1 = pl.pallas_call(
        _tc_pre_body,
        grid=(GRID,),
        in_specs=[_row_spec, _deg_spec, _vec_spec, _vec_spec, _mat_spec],
        **_row_out((N, D)),
    )(x, degt, ln0_w, ln0_b, W1)

    acc1 = _sc_edge_scatter(y1, sdg)

    y2 = pl.pallas_call(
        _tc_mid_body,
        grid=(GRID,),
        in_specs=[_acca_spec, _accb_spec, _row_spec, _deg_spec,
                  _vec_spec, _vec_spec, _vec_spec, _mat_spec],
        **_row_out((N, D)),
    )(acc1, acc1, y1, degt, b1, ln1_w, ln1_b, W2)

    acc2 = _sc_edge_scatter(y2, sdg)

    out = pl.pallas_call(
        _tc_post_body,
        grid=(GRID,),
        in_specs=[_acca_spec, _accb_spec, _row_spec, _deg_spec,
                  _vec_spec, _vec_spec, _vec_spec],
        out_specs=pl.BlockSpec((1, D), lambda i: (0, 0)),
        out_shape=jax.ShapeDtypeStruct((1, D), jnp.float32),
    )(acc2, acc2, y2, degt, b2, ln2_w, ln2_b)

    return out


# speedup vs baseline: 31.3342x; 1.0006x over previous
import functools

import jax
import jax.numpy as jnp
from jax import lax
from jax.experimental import pallas as pl
from jax.experimental.pallas import tpu as pltpu
from jax.experimental.pallas import tpu_sc as plsc

N = 10000
E = 320000
D = 128

NC = 2
NS = 16
NW = NC * NS

EPW = E // NW
CHUNK = 80
NCHUNK = 125
NPAD = 10240
RPT = NPAD // NS

_mesh = plsc.VectorSubcoreMesh(core_axis_name="c", subcore_axis_name="s")


@functools.partial(
    pl.kernel,
    out_type=jax.ShapeDtypeStruct((NC, NS, N + 240), jnp.float32),
    mesh=_mesh,
    compiler_params=pltpu.CompilerParams(needs_layout_passes=False),
    scratch_types=[
        pltpu.VMEM((EPW,), jnp.int32),
        pltpu.VMEM((N + 240,), jnp.float32),
    ],
)
def _sc_degree(dst_hbm, out_hbm, dst_v, hist):
    c = lax.axis_index("c")
    s = lax.axis_index("s")
    wid = s * NC + c

    zero16 = jnp.zeros((16,), jnp.float32)

    def zb(j, carry):
        hist[pl.ds(j * 16, 16)] = zero16
        return carry

    lax.fori_loop(0, (N + 240) // 16, zb, 0)

    pltpu.sync_copy(dst_hbm.at[pl.ds(wid * EPW, EPW)], dst_v)

    ones16 = jnp.ones((16,), jnp.float32)

    def step(j, carry):
        idx = dst_v[pl.ds(j * 16, 16)]
        plsc.addupdate_scatter(hist, [idx], ones16)
        return carry

    lax.fori_loop(0, EPW // 16, step, 0)

    pltpu.sync_copy(hist, out_hbm.at[c, s])


@functools.partial(
    pl.kernel,
    out_type=jax.ShapeDtypeStruct((NC, NPAD, D), jnp.float32),
    mesh=_mesh,
    compiler_params=pltpu.CompilerParams(needs_layout_passes=False),
    scratch_types=[
        pltpu.VMEM((2, CHUNK), jnp.int32),
        pltpu.VMEM((2, CHUNK), jnp.int32),
        pltpu.VMEM((2, CHUNK), jnp.int32),
        pltpu.VMEM((CHUNK, D), jnp.float32),
        pltpu.VMEM((CHUNK, D), jnp.float32),
        pltpu.VMEM((CHUNK, D), jnp.float32),
        pltpu.VMEM_SHARED((NPAD, D), jnp.float32),
        pltpu.SemaphoreType.DMA,
        pltpu.SemaphoreType.DMA,
        pltpu.SemaphoreType.DMA,
        pltpu.SemaphoreType.DMA,
        pltpu.SemaphoreType.DMA,
        pltpu.SemaphoreType.DMA,
    ],
)
def _sc_edge_scatter(y_hbm, sdg_hbm, out_hbm,
                     sd0, sd1, sd2, rows0, rows1, rows2, acc,
                     seml0, seml1, seml2, semg0, semg1, semg2):
    c = lax.axis_index("c")
    s = lax.axis_index("s")
    wid = s * NC + c

    zero16 = jnp.zeros((16,), jnp.float32)

    def zb(j, carry):
        for k in range(8):
            rows0[j, pl.ds(k * 16, 16)] = zero16
        return carry

    lax.fori_loop(0, CHUNK, zb, 0)

    for m in range(RPT // 80):
        pltpu.sync_copy(rows0.at[pl.ds(0, 80)],
                        acc.at[pl.ds(s * RPT + m * 80, 80)])

    sds = (sd0, sd1, sd2)
    rowss = (rows0, rows1, rows2)
    semls = (seml0, seml1, seml2)
    semgs = (semg0, semg1, semg2)

    def idxload(i, b):
        pltpu.async_copy(sdg_hbm.at[wid, i], sds[b], semls[b])

    def lwait(b):
        pltpu.make_async_copy(sdg_hbm.at[wid, 0], sds[b], semls[b]).wait()

    def gather(i, b):
        pltpu.async_copy(y_hbm.at[sds[b].at[0]], rowss[b], semgs[b])

    def gwait(b):
        pltpu.make_async_copy(y_hbm.at[pl.ds(0, CHUNK)], rowss[b],
                              semgs[b]).wait()

    def scatter(b):
        pltpu.sync_copy(rowss[b], acc.at[sds[b].at[1]], add=True)

    idxload(0, 0)
    idxload(1, 1)
    idxload(2, 2)
    lwait(0)
    gather(0, 0)
    lwait(1)
    gather(1, 1)

    plsc.subcore_barrier()

    def chunk_steady(i, b):
        gwait(b)
        scatter(b)
        idxload(i + 3, b)
        b2 = (b + 2) % 3
        lwait(b2)
        gather(i + 2, b2)

    def step(m, carry):
        i = 3 * m
        chunk_steady(i, 0)
        chunk_steady(i + 1, 1)
        chunk_steady(i + 2, 2)
        return carry

    lax.fori_loop(0, (NCHUNK - 5) // 3, step, 0)

    t = NCHUNK - 5
    chunk_steady(t, 0)
    chunk_steady(t + 1, 1)
    gwait(2)
    scatter(2)
    lwait(1)
    gather(t + 4, 1)
    gwait(0)
    scatter(0)
    gwait(1)
    scatter(1)

    plsc.subcore_barrier()

    pltpu.sync_copy(acc.at[pl.ds(s * RPT, RPT)],
                    out_hbm.at[c, pl.ds(s * RPT, RPT)])


BLK = 5000
GRID = N // BLK


def _ln(h, w, b):
    mu = jnp.mean(h, axis=1, keepdims=True)
    var = jnp.mean((h - mu) * (h - mu), axis=1, keepdims=True)
    return (h - mu) * lax.rsqrt(var + 1e-5) * w + b


def _dinv(degt):
    return lax.rsqrt(jnp.sum(degt, axis=1, keepdims=True) + 1.0)


def _tc_pre_body(x_ref, degt_ref, lnw_ref, lnb_ref, w_ref, o_ref):
    h = _ln(x_ref[...], lnw_ref[...], lnb_ref[...])
    o_ref[...] = _dinv(degt_ref[...]) * jnp.dot(
        h, w_ref[...], preferred_element_type=jnp.float32)


def _tc_mid_body(acca_ref, accb_ref, y_ref, degt_ref, b_ref,
                 lnw_ref, lnb_ref, w_ref, o_ref):
    dinv = _dinv(degt_ref[...])
    t = dinv * (acca_ref[0] + accb_ref[0] + y_ref[...]) + b_ref[...]
    t = jnp.maximum(t, 0.0)
    h = _ln(t, lnw_ref[...], lnb_ref[...])
    o_ref[...] = dinv * jnp.dot(h, w_ref[...],
                                preferred_element_type=jnp.float32)


def _tc_post_body(acca_ref, accb_ref, y_ref, degt_ref, b_ref,
                  lnw_ref, lnb_ref, o_ref):
    dinv = _dinv(degt_ref[...])
    t = dinv * (acca_ref[0] + accb_ref[0] + y_ref[...]) + b_ref[...]
    t = jnp.maximum(t, 0.0)
    h = _ln(t, lnw_ref[...], lnb_ref[...])

    i = pl.program_id(0)

    @pl.when(i == 0)
    def _():
        o_ref[...] = jnp.zeros_like(o_ref)

    o_ref[...] += jnp.sum(h, axis=0, keepdims=True) * (1.0 / N)


_row_spec = pl.BlockSpec((BLK, D), lambda i: (i, 0))
_deg_spec = pl.BlockSpec((BLK, NW), lambda i: (i, 0))
_vec_spec = pl.BlockSpec((1, D), lambda i: (0, 0))
_mat_spec = pl.BlockSpec((D, D), lambda i: (0, 0))
_acca_spec = pl.BlockSpec((1, BLK, D), lambda i: (0, i, 0))
_accb_spec = pl.BlockSpec((1, BLK, D), lambda i: (1, i, 0))


def _row_out(shape):
    return dict(out_specs=_row_spec,
                out_shape=jax.ShapeDtypeStruct(shape, jnp.float32))


def kernel(x, edge_index, ln0_w, ln0_b, W1, b1, ln1_w, ln1_b, W2, b2,
           ln2_w, ln2_b):
    dst = edge_index[1]
    sdg = edge_index.reshape(2, NW, NCHUNK, CHUNK).transpose(1, 2, 0, 3)

    ln0_w = ln0_w.reshape(1, D)
    ln0_b = ln0_b.reshape(1, D)
    ln1_w = ln1_w.reshape(1, D)
    ln1_b = ln1_b.reshape(1, D)
    ln2_w = ln2_w.reshape(1, D)
    ln2_b = ln2_b.reshape(1, D)
    b1 = b1.reshape(1, D)
    b2 = b2.reshape(1, D)

    deg_parts = _sc_degree(dst)
    degt = deg_parts.reshape(NW, N + 240).T[:N]

    y1 = pl.pallas_call(
        _tc_pre_body,
        grid=(GRID,),
        in_specs=[_row_spec, _deg_spec, _vec_spec, _vec_spec, _mat_spec],
        **_row_out((N, D)),
    )(x, degt, ln0_w, ln0_b, W1)

    acc1 = _sc_edge_scatter(y1, sdg)

    y2 = pl.pallas_call(
        _tc_mid_body,
        grid=(GRID,),
        in_specs=[_acca_spec, _accb_spec, _row_spec, _deg_spec,
                  _vec_spec, _vec_spec, _vec_spec, _mat_spec],
        **_row_out((N, D)),
    )(acc1, acc1, y1, degt, b1, ln1_w, ln1_b, W2)

    acc2 = _sc_edge_scatter(y2, sdg)

    out = pl.pallas_call(
        _tc_post_body,
        grid=(GRID,),
        in_specs=[_acca_spec, _accb_spec, _row_spec, _deg_spec,
                  _vec_spec, _vec_spec, _vec_spec],
        out_specs=pl.BlockSpec((1, D), lambda i: (0, 0)),
        out_shape=jax.ShapeDtypeStruct((1, D), jnp.float32),
    )(acc2, acc2, y2, degt, b2, ln2_w, ln2_b)

    return out
